# Initial kernel scaffold; baseline (speedup 1.0000x reference)
import jax
import jax.numpy as jnp
from jax.experimental import pallas as pl


def kernel(in_feat, edge_index, W1, b1, W2, b2):
    raise NotImplementedError("write your pallas kernel here")


# speedup vs baseline: 3.9945x; 3.9945x over previous
import functools

import jax
import jax.numpy as jnp
from jax import lax
from jax.experimental import pallas as pl
from jax.experimental.pallas import tpu as pltpu
from jax.experimental.pallas import tpu_sc as plsc

N = 10000
E = 320000
D_IN = 128
D_H = 128
D_OUT = 64

NPAD = 10240
DUMMY = N
NW = 32
C = 128
CHUNKS = -(-E // (NW * C))
EPW = CHUNKS * C
ROWS_PER_TILE = NPAD // 16

_MESH = plsc.VectorSubcoreMesh(core_axis_name="c", subcore_axis_name="s")


@functools.partial(
    pl.kernel,
    out_type=jax.ShapeDtypeStruct((2, NPAD, D_H), jnp.float32),
    mesh=_MESH,
    scratch_types=[
        pltpu.VMEM((CHUNKS, C), jnp.int32),
        pltpu.VMEM((C, D_H), jnp.float32),
        pltpu.VMEM_SHARED((NPAD, D_H), jnp.float32),
    ],
)
def _deg_kernel(slabs_hbm, ones_hbm, zeros_hbm, degp_hbm, idx, ones_v, acc):
    c = lax.axis_index("c")
    s = lax.axis_index("s")
    pltpu.sync_copy(ones_hbm, ones_v)
    r0 = s * ROWS_PER_TILE
    pltpu.sync_copy(zeros_hbm.at[pl.ds(r0, ROWS_PER_TILE)],
                    acc.at[pl.ds(r0, ROWS_PER_TILE)])
    plsc.subcore_barrier()

    def outer(k, carry):
        pltpu.sync_copy(slabs_hbm.at[c, s * 2 + k], idx)

        def body(j, cc):
            pltpu.sync_copy(ones_v, acc.at[idx.at[j]], add=True)
            return cc

        return lax.fori_loop(0, CHUNKS, body, carry)

    lax.fori_loop(0, 2, outer, 0)
    plsc.subcore_barrier()
    pltpu.sync_copy(acc.at[pl.ds(r0, ROWS_PER_TILE)],
                    degp_hbm.at[c, pl.ds(r0, ROWS_PER_TILE)])


def _make_agg_kernel(D):
    @functools.partial(
        pl.kernel,
        out_type=jax.ShapeDtypeStruct((2, NPAD, D), jnp.float32),
        mesh=_MESH,
        scratch_types=[
            pltpu.VMEM((CHUNKS, C), jnp.int32),
            pltpu.VMEM((CHUNKS, C), jnp.int32),
            pltpu.VMEM((C, D), jnp.float32),
            pltpu.VMEM_SHARED((NPAD, D), jnp.float32),
            pltpu.SemaphoreType.DMA,
        ],
    )
    def _agg(h_hbm, src_hbm, dst_hbm, zeros_hbm, out_hbm,
             idx_src, idx_dst, rows, acc, sem):
        c = lax.axis_index("c")
        s = lax.axis_index("s")
        w = c * 16 + s
        pltpu.sync_copy(src_hbm.at[w], idx_src)
        pltpu.sync_copy(dst_hbm.at[w], idx_dst)
        r0 = s * ROWS_PER_TILE
        pltpu.sync_copy(zeros_hbm.at[pl.ds(r0, ROWS_PER_TILE)],
                        acc.at[pl.ds(r0, ROWS_PER_TILE)])
        plsc.subcore_barrier()

        def body(j, carry):
            pltpu.async_copy(h_hbm.at[idx_src.at[j]], rows, sem).wait()
            pltpu.sync_copy(rows, acc.at[idx_dst.at[j]], add=True)
            return carry

        lax.fori_loop(0, CHUNKS, body, 0)
        plsc.subcore_barrier()
        pltpu.sync_copy(acc.at[pl.ds(r0, ROWS_PER_TILE)],
                        out_hbm.at[c, pl.ds(r0, ROWS_PER_TILE)])

    return _agg


_agg128 = _make_agg_kernel(D_H)


BLK = 1024
GRID = NPAD // BLK


def _norms(degp):
    ns = lax.rsqrt(jnp.maximum(degp[0], 1.0))[:, 0:1]
    nd = lax.rsqrt(jnp.maximum(degp[1], 1.0))[:, 0:1]
    return ns, nd


def _tc_a_body(x_ref, degp_ref, w1_ref, o_ref):
    ns, _ = _norms(degp_ref[...])
    o_ref[...] = jnp.dot(x_ref[...] * ns, w1_ref[...],
                         preferred_element_type=jnp.float32)


def _tc_b_body(agg_ref, degp_ref, b1_ref, w2_ref, o_ref):
    ns, nd = _norms(degp_ref[...])
    a = agg_ref[0] + agg_ref[1]
    h = jax.nn.relu(a * nd + b1_ref[...])
    o_ref[...] = jnp.dot(h * ns, w2_ref[...],
                         preferred_element_type=jnp.float32)


def _tc_c_body(agg_ref, degp_ref, b2_ref, o_ref):
    _, nd = _norms(degp_ref[...])
    a = agg_ref[0] + agg_ref[1]
    o_ref[...] = a * nd + b2_ref[...]


_degp_spec = pl.BlockSpec((2, BLK, D_H), lambda i: (0, i, 0))


def _tc_a(x, degp, w1):
    return pl.pallas_call(
        _tc_a_body,
        grid=(GRID,),
        in_specs=[
            pl.BlockSpec((BLK, D_IN), lambda i: (i, 0)),
            _degp_spec,
            pl.BlockSpec((D_IN, D_H), lambda i: (0, 0)),
        ],
        out_specs=pl.BlockSpec((BLK, D_H), lambda i: (i, 0)),
        out_shape=jax.ShapeDtypeStruct((NPAD, D_H), jnp.float32),
    )(x, degp, w1)


def _tc_b(agg1, degp, b1, w2p):
    return pl.pallas_call(
        _tc_b_body,
        grid=(GRID,),
        in_specs=[
            pl.BlockSpec((2, BLK, D_H), lambda i: (0, i, 0)),
            _degp_spec,
            pl.BlockSpec((1, D_H), lambda i: (0, 0)),
            pl.BlockSpec((D_H, D_H), lambda i: (0, 0)),
        ],
        out_specs=pl.BlockSpec((BLK, D_H), lambda i: (i, 0)),
        out_shape=jax.ShapeDtypeStruct((NPAD, D_H), jnp.float32),
    )(agg1, degp, b1, w2p)


def _tc_c(agg2, degp, b2):
    return pl.pallas_call(
        _tc_c_body,
        grid=(GRID,),
        in_specs=[
            pl.BlockSpec((2, BLK, D_H), lambda i: (0, i, 0)),
            _degp_spec,
            pl.BlockSpec((1, D_H), lambda i: (0, 0)),
        ],
        out_specs=pl.BlockSpec((BLK, D_H), lambda i: (i, 0)),
        out_shape=jax.ShapeDtypeStruct((NPAD, D_H), jnp.float32),
    )(agg2, degp, b2)


def kernel(in_feat, edge_index, W1, b1, W2, b2):
    src = edge_index[0]
    dst = edge_index[1]
    pad = NW * EPW - E
    fill = jnp.full((pad,), DUMMY, jnp.int32)
    srcp = jnp.concatenate([src, fill]).reshape(NW, CHUNKS, C)
    dstp = jnp.concatenate([dst, fill]).reshape(NW, CHUNKS, C)
    x_pad = jnp.pad(in_feat, ((0, NPAD - N), (0, 0)))

    ones128 = jnp.ones((C, D_H), jnp.float32)
    zeros128 = jnp.zeros((NPAD, D_H), jnp.float32)
    w2p = jnp.pad(W2, ((0, 0), (0, D_H - D_OUT)))
    b2p = jnp.pad(b2, (0, D_H - D_OUT)).reshape(1, D_H)

    slabs = jnp.stack([srcp, dstp])
    degp = _deg_kernel(slabs, ones128, zeros128)
    hs1 = _tc_a(x_pad, degp, W1)
    agg1 = _agg128(hs1, srcp, dstp, zeros128)
    hs2 = _tc_b(agg1, degp, b1.reshape(1, D_H), w2p)
    agg2 = _agg128(hs2, srcp, dstp, zeros128)
    outp = _tc_c(agg2, degp, b2p)
    return outp[:N, :D_OUT]


# speedup vs baseline: 3.9975x; 1.0007x over previous
import functools

import jax
import jax.numpy as jnp
from jax import lax
from jax.experimental import pallas as pl
from jax.experimental.pallas import tpu as pltpu
from jax.experimental.pallas import tpu_sc as plsc

N = 10000
E = 320000
D_IN = 128
D_H = 128
D_OUT = 64

NPAD = 10240
DUMMY = N
NW = 32
C = 96
HCH = -(-E // (NW * C * 2))
CHUNKS = 2 * HCH
EPW = CHUNKS * C
ROWS_PER_TILE = NPAD // 16

_MESH = plsc.VectorSubcoreMesh(core_axis_name="c", subcore_axis_name="s")


@functools.partial(
    pl.kernel,
    out_type=jax.ShapeDtypeStruct((2, NPAD, D_H), jnp.float32),
    mesh=_MESH,
    scratch_types=[
        pltpu.VMEM((2, 2, HCH, C), jnp.int32),
        pltpu.VMEM((C, D_H), jnp.float32),
        pltpu.VMEM_SHARED((NPAD, D_H), jnp.float32),
        pltpu.SemaphoreType.DMA,
    ],
)
def _deg_kernel(slabs_hbm, ones_hbm, zeros_hbm, degp_hbm, idx, ones_v, acc, sem):
    c = lax.axis_index("c")
    s = lax.axis_index("s")
    pltpu.sync_copy(ones_hbm, ones_v)
    pltpu.sync_copy(slabs_hbm.at[c, pl.ds(s * 2, 2)], idx)
    r0 = s * ROWS_PER_TILE
    pltpu.sync_copy(zeros_hbm.at[pl.ds(r0, ROWS_PER_TILE)],
                    acc.at[pl.ds(r0, ROWS_PER_TILE)])
    plsc.subcore_barrier()

    def fire(j, carry):
        for k in range(2):
            for h in range(2):
                pltpu.async_copy(ones_v, acc.at[idx.at[k, h, j]], sem, add=True)
        return carry

    lax.fori_loop(0, HCH, fire, 0)

    def drain(j, carry):
        for k in range(2):
            for h in range(2):
                pltpu.make_async_copy(ones_v, acc.at[idx.at[k, h, j]], sem).wait()
        return carry

    lax.fori_loop(0, HCH, drain, 0)
    plsc.subcore_barrier()
    pltpu.sync_copy(acc.at[pl.ds(r0, ROWS_PER_TILE)],
                    degp_hbm.at[c, pl.ds(r0, ROWS_PER_TILE)])


def _make_agg_kernel(D):
    @functools.partial(
        pl.kernel,
        out_type=jax.ShapeDtypeStruct((2, NPAD, D), jnp.float32),
        mesh=_MESH,
        scratch_types=[
            pltpu.VMEM((HCH, C), jnp.int32),
            pltpu.VMEM((HCH, C), jnp.int32),
            pltpu.VMEM((C, D), jnp.float32),
            pltpu.VMEM((C, D), jnp.float32),
            pltpu.VMEM_SHARED((NPAD, D), jnp.float32),
            pltpu.SemaphoreType.DMA,
            pltpu.SemaphoreType.DMA,
            pltpu.SemaphoreType.DMA,
            pltpu.SemaphoreType.DMA,
        ],
    )
    def _agg(h_hbm, src_hbm, dst_hbm, zeros_hbm, out_hbm,
             idx_src, idx_dst, rows0, rows1, acc, gsem0, gsem1, ssem0, ssem1):
        c = lax.axis_index("c")
        s = lax.axis_index("s")
        w = c * 16 + s
        r0 = s * ROWS_PER_TILE
        pltpu.sync_copy(zeros_hbm.at[pl.ds(r0, ROWS_PER_TILE)],
                        acc.at[pl.ds(r0, ROWS_PER_TILE)])
        plsc.subcore_barrier()

        bufs = (rows0, rows1)
        gsems = (gsem0, gsem1)
        ssems = (ssem0, ssem1)

        def run_half(half, n):
            pltpu.sync_copy(src_hbm.at[w, half], idx_src)
            pltpu.sync_copy(dst_hbm.at[w, half], idx_dst)
            pltpu.async_copy(h_hbm.at[idx_src.at[0]], rows0, gsem0)

            def step(j, cur, nxt):
                pltpu.make_async_copy(h_hbm.at[idx_src.at[j]], bufs[cur],
                                      gsems[cur]).wait()

                @pl.when(j >= 1)
                def _():
                    pltpu.make_async_copy(bufs[nxt], acc.at[idx_dst.at[j - 1]],
                                          ssems[nxt]).wait()

                @pl.when(j + 1 < n)
                def _():
                    pltpu.async_copy(h_hbm.at[idx_src.at[j + 1]], bufs[nxt],
                                     gsems[nxt])

                pltpu.async_copy(bufs[cur], acc.at[idx_dst.at[j]], ssems[cur],
                                 add=True)

            def body(j, carry):
                @pl.when(j % 2 == 0)
                def _():
                    step(j, 0, 1)

                @pl.when(j % 2 == 1)
                def _():
                    step(j, 1, 0)

                return carry

            lax.fori_loop(0, n, body, 0)
            last = (n - 1) % 2
            pltpu.make_async_copy(bufs[last], acc.at[idx_dst.at[n - 1]],
                                  ssems[last]).wait()

        run_half(0, HCH)
        run_half(1, HCH)
        plsc.subcore_barrier()
        pltpu.sync_copy(acc.at[pl.ds(r0, ROWS_PER_TILE)],
                        out_hbm.at[c, pl.ds(r0, ROWS_PER_TILE)])

    return _agg


_agg128 = _make_agg_kernel(D_H)


BLK = 1024
GRID = NPAD // BLK


def _norms(degp):
    ns = lax.rsqrt(jnp.maximum(degp[0], 1.0))[:, 0:1]
    nd = lax.rsqrt(jnp.maximum(degp[1], 1.0))[:, 0:1]
    return ns, nd


def _tc_a_body(x_ref, degp_ref, w1_ref, o_ref):
    ns, _ = _norms(degp_ref[...])
    o_ref[...] = jnp.dot(x_ref[...] * ns, w1_ref[...],
                         preferred_element_type=jnp.float32)


def _tc_b_body(agg_ref, degp_ref, b1_ref, w2_ref, o_ref):
    ns, nd = _norms(degp_ref[...])
    a = agg_ref[0] + agg_ref[1]
    h = jax.nn.relu(a * nd + b1_ref[...])
    o_ref[...] = jnp.dot(h * ns, w2_ref[...],
                         preferred_element_type=jnp.float32)


def _tc_c_body(agg_ref, degp_ref, b2_ref, o_ref):
    _, nd = _norms(degp_ref[...])
    a = agg_ref[0] + agg_ref[1]
    o_ref[...] = a * nd + b2_ref[...]


_degp_spec = pl.BlockSpec((2, BLK, D_H), lambda i: (0, i, 0))


def _tc_a(x, degp, w1):
    return pl.pallas_call(
        _tc_a_body,
        grid=(GRID,),
        in_specs=[
            pl.BlockSpec((BLK, D_IN), lambda i: (i, 0)),
            _degp_spec,
            pl.BlockSpec((D_IN, D_H), lambda i: (0, 0)),
        ],
        out_specs=pl.BlockSpec((BLK, D_H), lambda i: (i, 0)),
        out_shape=jax.ShapeDtypeStruct((NPAD, D_H), jnp.float32),
    )(x, degp, w1)


def _tc_b(agg1, degp, b1, w2p):
    return pl.pallas_call(
        _tc_b_body,
        grid=(GRID,),
        in_specs=[
            pl.BlockSpec((2, BLK, D_H), lambda i: (0, i, 0)),
            _degp_spec,
            pl.BlockSpec((1, D_H), lambda i: (0, 0)),
            pl.BlockSpec((D_H, D_H), lambda i: (0, 0)),
        ],
        out_specs=pl.BlockSpec((BLK, D_H), lambda i: (i, 0)),
        out_shape=jax.ShapeDtypeStruct((NPAD, D_H), jnp.float32),
    )(agg1, degp, b1, w2p)


def _tc_c(agg2, degp, b2):
    return pl.pallas_call(
        _tc_c_body,
        grid=(GRID,),
        in_specs=[
            pl.BlockSpec((2, BLK, D_H), lambda i: (0, i, 0)),
            _degp_spec,
            pl.BlockSpec((1, D_H), lambda i: (0, 0)),
        ],
        out_specs=pl.BlockSpec((BLK, D_H), lambda i: (i, 0)),
        out_shape=jax.ShapeDtypeStruct((NPAD, D_H), jnp.float32),
    )(agg2, degp, b2)


def kernel(in_feat, edge_index, W1, b1, W2, b2):
    src = edge_index[0]
    dst = edge_index[1]
    pad = NW * EPW - E
    fill = jnp.full((pad,), DUMMY, jnp.int32)
    srcp = jnp.concatenate([src, fill]).reshape(NW, 2, HCH, C)
    dstp = jnp.concatenate([dst, fill]).reshape(NW, 2, HCH, C)
    x_pad = jnp.pad(in_feat, ((0, NPAD - N), (0, 0)))

    ones128 = jnp.ones((C, D_H), jnp.float32)
    zeros128 = jnp.zeros((NPAD, D_H), jnp.float32)
    w2p = jnp.pad(W2, ((0, 0), (0, D_H - D_OUT)))
    b2p = jnp.pad(b2, (0, D_H - D_OUT)).reshape(1, D_H)

    slabs = jnp.stack([srcp, dstp])
    degp = _deg_kernel(slabs, ones128, zeros128)
    hs1 = _tc_a(x_pad, degp, W1)
    agg1 = _agg128(hs1, srcp, dstp, zeros128)
    hs2 = _tc_b(agg1, degp, b1.reshape(1, D_H), w2p)
    agg2 = _agg128(hs2, srcp, dstp, zeros128)
    outp = _tc_c(agg2, degp, b2p)
    return outp[:N, :D_OUT]
